# baseline (speedup 1.0000x reference)
import functools

import jax
import jax.numpy as jnp
from jax import lax
from jax.experimental import pallas as pl
from jax.experimental.pallas import tpu as pltpu
from jax.experimental.pallas import tpu_sc as plsc

B = 4
SEQ = 2048
NVIS = 256
HIDDEN = 1024

NC = 2
NS = 16
NW = NC * NS
TOK_PER_W = (B * SEQ) // NW
CHUNKS = (32, 56, 64, 56, 48)
BUF0_ROWS = max(CHUNKS[0::2])
BUF1_ROWS = max(CHUNKS[1::2])
NCHUNK = len(CHUNKS)


_SC_MESH = plsc.VectorSubcoreMesh(core_axis_name="c", subcore_axis_name="s")


@functools.partial(
    pl.kernel,
    mesh=_SC_MESH,
    out_type=jax.ShapeDtypeStruct((B, NVIS + SEQ, HIDDEN), jnp.float32),
    scratch_types=[
        pltpu.VMEM((TOK_PER_W,), jnp.int32),
        pltpu.VMEM((BUF0_ROWS, HIDDEN), jnp.float32),
        pltpu.VMEM((BUF1_ROWS, HIDDEN), jnp.float32),
        pltpu.SemaphoreType.DMA,
        pltpu.SemaphoreType.DMA,
    ],
)
def _gather_txt(tok_hbm, table_hbm, out_hbm, idx_v, buf0, buf1, gsem, ssem):
    wid = lax.axis_index("s") * NC + lax.axis_index("c")
    b = wid // (NW // B)
    part = wid % (NW // B)

    pltpu.sync_copy(tok_hbm.at[b, pl.ds(part * TOK_PER_W, TOK_PER_W)], idx_v)

    bufs = (buf0, buf1)
    offs = [sum(CHUNKS[:j]) for j in range(NCHUNK)]
    s0 = NVIS + part * TOK_PER_W

    def _buf(j):
        ch = CHUNKS[j]
        buf = bufs[j % 2]
        return buf if ch == buf.shape[0] else buf.at[pl.ds(0, ch), :]

    def _gather(j):
        return pltpu.async_copy(
            table_hbm.at[idx_v.at[pl.ds(offs[j], CHUNKS[j])]], _buf(j), gsem)

    g_h = [None] * NCHUNK
    s_h = [None] * NCHUNK
    g_h[0] = _gather(0)
    for j in range(NCHUNK):
        ch = CHUNKS[j]
        g_h[j].wait()
        if j + 1 < NCHUNK:
            if j >= 1:
                s_h[j - 1].wait()
            g_h[j + 1] = _gather(j + 1)
        s_h[j] = pltpu.async_copy(
            _buf(j), out_hbm.at[b, pl.ds(s0 + offs[j], ch), :], ssem)
    s_h[NCHUNK - 2].wait()
    s_h[NCHUNK - 1].wait()


def _proj_body(x_ref, w_ref, b_ref, o_ref):
    o_ref[0] = lax.dot_general(
        x_ref[0], w_ref[...], (((1,), (1,)), ((), ())),
        preferred_element_type=jnp.float32,
    ) + b_ref[...]


def _project(visual, w, b_lin):
    return pl.pallas_call(
        _proj_body,
        grid=(B,),
        in_specs=[
            pl.BlockSpec((1, NVIS, HIDDEN), lambda i: (i, 0, 0)),
            pl.BlockSpec((HIDDEN, HIDDEN), lambda i: (0, 0)),
            pl.BlockSpec((HIDDEN,), lambda i: (0,)),
        ],
        out_specs=pl.BlockSpec((1, NVIS, HIDDEN), lambda i: (i, 0, 0)),
        out_shape=jax.ShapeDtypeStruct((B, NVIS, HIDDEN), jnp.float32),
    )(visual, w, b_lin)


def _stitch_body(big_ref, vis_ref, o_ref):
    del big_ref
    o_ref[...] = vis_ref[...]


def _stitch(big, vis):
    return pl.pallas_call(
        _stitch_body,
        grid=(2,),
        in_specs=[
            pl.BlockSpec(memory_space=pl.ANY),
            pl.BlockSpec((B // 2, NVIS, HIDDEN), lambda i: (i, 0, 0)),
        ],
        out_specs=pl.BlockSpec((B // 2, NVIS, HIDDEN), lambda i: (i, 0, 0)),
        out_shape=jax.ShapeDtypeStruct((B, NVIS + SEQ, HIDDEN), jnp.float32),
        input_output_aliases={0: 0},
    )(big, vis)


def kernel(text_tokens, visual_embeddings, W, b_lin, table):
    big = _gather_txt(text_tokens, table)
    vis = _project(visual_embeddings, W, b_lin)
    return _stitch(big, vis)


# speedup vs baseline: 1.0175x; 1.0175x over previous
import functools

import jax
import jax.numpy as jnp
from jax import lax
from jax.experimental import pallas as pl
from jax.experimental.pallas import tpu as pltpu
from jax.experimental.pallas import tpu_sc as plsc

B = 4
SEQ = 2048
NVIS = 256
HIDDEN = 1024

NC = 2
NS = 16
NW = NC * NS
TOK_PER_W = (B * SEQ) // NW
CHUNKS = (56, 56, 56, 56, 32)
BUF0_ROWS = max(CHUNKS[0::2])
BUF1_ROWS = max(CHUNKS[1::2])
NCHUNK = len(CHUNKS)


_SC_MESH = plsc.VectorSubcoreMesh(core_axis_name="c", subcore_axis_name="s")


@functools.partial(
    pl.kernel,
    mesh=_SC_MESH,
    out_type=jax.ShapeDtypeStruct((B, NVIS + SEQ, HIDDEN), jnp.float32),
    scratch_types=[
        pltpu.VMEM((TOK_PER_W,), jnp.int32),
        pltpu.VMEM((BUF0_ROWS, HIDDEN), jnp.float32),
        pltpu.VMEM((BUF1_ROWS, HIDDEN), jnp.float32),
        pltpu.SemaphoreType.DMA,
        pltpu.SemaphoreType.DMA,
    ],
)
def _gather_txt(tok_hbm, table_hbm, out_hbm, idx_v, buf0, buf1, gsem, ssem):
    wid = lax.axis_index("s") * NC + lax.axis_index("c")
    b = wid // (NW // B)
    part = wid % (NW // B)

    pltpu.sync_copy(tok_hbm.at[b, pl.ds(part * TOK_PER_W, TOK_PER_W)], idx_v)

    bufs = (buf0, buf1)
    offs = [sum(CHUNKS[:j]) for j in range(NCHUNK)]
    s0 = NVIS + part * TOK_PER_W

    def _buf(j):
        ch = CHUNKS[j]
        buf = bufs[j % 2]
        return buf if ch == buf.shape[0] else buf.at[pl.ds(0, ch), :]

    def _gather(j):
        return pltpu.async_copy(
            table_hbm.at[idx_v.at[pl.ds(offs[j], CHUNKS[j])]], _buf(j), gsem)

    g_h = [None] * NCHUNK
    s_h = [None] * NCHUNK
    g_h[0] = _gather(0)
    for j in range(NCHUNK):
        ch = CHUNKS[j]
        g_h[j].wait()
        if j + 1 < NCHUNK:
            if j >= 1:
                s_h[j - 1].wait()
            g_h[j + 1] = _gather(j + 1)
        s_h[j] = pltpu.async_copy(
            _buf(j), out_hbm.at[b, pl.ds(s0 + offs[j], ch), :], ssem)
    s_h[NCHUNK - 2].wait()
    s_h[NCHUNK - 1].wait()


def _proj_body(x_ref, w_ref, b_ref, o_ref):
    o_ref[0] = lax.dot_general(
        x_ref[0], w_ref[...], (((1,), (1,)), ((), ())),
        preferred_element_type=jnp.float32,
    ) + b_ref[...]


def _project(visual, w, b_lin):
    return pl.pallas_call(
        _proj_body,
        grid=(B,),
        in_specs=[
            pl.BlockSpec((1, NVIS, HIDDEN), lambda i: (i, 0, 0)),
            pl.BlockSpec((HIDDEN, HIDDEN), lambda i: (0, 0)),
            pl.BlockSpec((HIDDEN,), lambda i: (0,)),
        ],
        out_specs=pl.BlockSpec((1, NVIS, HIDDEN), lambda i: (i, 0, 0)),
        out_shape=jax.ShapeDtypeStruct((B, NVIS, HIDDEN), jnp.float32),
    )(visual, w, b_lin)


def _stitch_body(big_ref, vis_ref, o_ref):
    del big_ref
    o_ref[...] = vis_ref[...]


def _stitch(big, vis):
    return pl.pallas_call(
        _stitch_body,
        grid=(2,),
        in_specs=[
            pl.BlockSpec(memory_space=pl.ANY),
            pl.BlockSpec((B // 2, NVIS, HIDDEN), lambda i: (i, 0, 0)),
        ],
        out_specs=pl.BlockSpec((B // 2, NVIS, HIDDEN), lambda i: (i, 0, 0)),
        out_shape=jax.ShapeDtypeStruct((B, NVIS + SEQ, HIDDEN), jnp.float32),
        input_output_aliases={0: 0},
    )(big, vis)


def kernel(text_tokens, visual_embeddings, W, b_lin, table):
    big = _gather_txt(text_tokens, table)
    vis = _project(visual_embeddings, W, b_lin)
    return _stitch(big, vis)
